# baseline (speedup 1.0000x reference)
import functools

import jax
import jax.numpy as jnp
from jax import lax
from jax.experimental import pallas as pl
from jax.experimental.pallas import tpu as pltpu
from jax.experimental.pallas import tpu_sc as plsc

_B = 8
_N = 128
_H = 32
_V = 1025
_D = 5
_F = 3
_K = _D * _F
_NPOS = _B * _N * _N
_NW = 32
_P = 128
_NCH = _NPOS // (_NW * _P)
_CH = 2048


def _tables_body(ee_ref, w5_ref, t_ref):
    row = lax.broadcasted_iota(jnp.int32, (_V, _H), 0)
    ee0 = jnp.where(row == 0, 0.0, ee_ref[...])
    for d in range(_D):
        t_ref[d] = jnp.dot(
            ee0, w5_ref[d], preferred_element_type=jnp.float32
        ).astype(jnp.bfloat16)


_tables_call = pl.pallas_call(
    _tables_body,
    out_shape=jax.ShapeDtypeStruct((_D, _V, _H), jnp.bfloat16),
)


_HW = _H // 2
_CPB = _NCH // 2


def _sc_body(t_hbm, idx_hbm, out_hbm, table_v, idx_v, acc_v, sem):
    wid = lax.axis_index("s") * 2 + lax.axis_index("c")
    pltpu.sync_copy(t_hbm, table_v)
    iota = lax.broadcasted_iota(jnp.int32, (16,), 0)

    def half_body(half, _):
        pltpu.sync_copy(idx_hbm.at[wid, pl.ds(half * _CPB * _K, _CPB * _K)],
                        idx_v)

        def chunk_body(c, _):
            def group_body(g, _):
                sl = pl.ds(g * 16, 16)
                rows = [
                    idx_v[c * _K + k, sl] + (k // _F) * _V
                    for k in range(_K)
                ]
                for c2 in range(_HW):
                    cols = jnp.full((16,), c2, jnp.int32)
                    acc = plsc.bitcast(
                        plsc.load_gather(table_v, [rows[0], cols]),
                        jnp.bfloat16,
                    )
                    for k in range(1, _K):
                        acc = acc + plsc.bitcast(
                            plsc.load_gather(table_v, [rows[k], cols]),
                            jnp.bfloat16,
                        )
                    plsc.store_scatter(
                        acc_v, [g * 16 + iota, cols],
                        plsc.bitcast(acc, jnp.int32),
                    )
                return 0

            lax.fori_loop(0, _P // 16, group_body, 0)
            base = wid * _NCH * _P + (half * _CPB + c) * _P
            pltpu.sync_copy(acc_v, out_hbm.at[pl.ds(base, _P)])
            return 0

        lax.fori_loop(0, _CPB, chunk_body, 0)
        return 0

    lax.fori_loop(0, 2, half_body, 0)


@functools.cache
def _sc_call():
    return pl.kernel(
        _sc_body,
        out_type=jax.ShapeDtypeStruct((_NPOS, _HW), jnp.int32),
        mesh=plsc.VectorSubcoreMesh(
            core_axis_name="c", subcore_axis_name="s",
            num_cores=2, num_subcores=16,
        ),
        scratch_types=[
            pltpu.VMEM((_D * _V, _HW), jnp.int32),
            pltpu.VMEM((_CPB * _K, _P), jnp.int32),
            pltpu.VMEM((_P, _HW), jnp.int32),
            pltpu.SemaphoreType.DMA,
        ],
        compiler_params=pltpu.CompilerParams(
            use_tc_tiling_on_sc=False, needs_layout_passes=False
        ),
    )


def _finish_body(ab_ref, sp_ref, eb_ref, out_ref):
    spi = sp_ref[0]
    spi = jnp.where(spi == 0, 1, spi)
    spi = jnp.where(spi > 1, spi - 1, spi)
    spf = jnp.clip(spi, 0, _D).astype(jnp.float32)
    scale = 1.0 / (3.0 * spf)
    eye = (
        lax.broadcasted_iota(jnp.int32, (_H, _H), 0)
        == lax.broadcasted_iota(jnp.int32, (_H, _H), 1)
    ).astype(jnp.bfloat16)
    ebt = lax.dot_general(
        eye, eb_ref[0], (((1,), (1,)), ((), ())),
        preferred_element_type=jnp.float32,
    )
    out_ref[0] = ab_ref[0] + ebt * scale


_NBLK = _NPOS // _CH

_finish_call = pl.pallas_call(
    _finish_body,
    grid=(_B, _N * _N // _CH),
    in_specs=[
        pl.BlockSpec((1, 1, _CH), lambda b, c: (b * (_N * _N // _CH) + c, 0, 0)),
        pl.BlockSpec((1, 1, _CH), lambda b, c: (b * (_N * _N // _CH) + c, 0, 0)),
        pl.BlockSpec((1, _CH, _H), lambda b, c: (b * (_N * _N // _CH) + c, 0, 0)),
    ],
    out_specs=pl.BlockSpec((1, _H, _CH), lambda b, c: (b, 0, c)),
    out_shape=jax.ShapeDtypeStruct((_B, _H, _N * _N), jnp.float32),
)


def kernel(attn_bias, spatial_pos, x, attn_edge_type, edge_input,
           edge_encoder_weight, edge_dis_encoder_weight):
    del x, attn_edge_type
    w5 = edge_dis_encoder_weight[: _D * _H * _H].reshape(_D, _H, _H)
    t = _tables_call(edge_encoder_weight, w5).reshape(_D * _V, _HW, 2)
    t_packed = lax.bitcast_convert_type(t, jnp.int32)
    idx4 = (
        edge_input.reshape(_NW, _NCH, _P, _K)
        .transpose(0, 1, 3, 2)
        .astype(jnp.int32)
        .reshape(_NW, _NCH * _K, _P)
    )
    eb = _sc_call()(t_packed, idx4)
    eb_bf = lax.bitcast_convert_type(eb, jnp.bfloat16)
    out = _finish_call(
        attn_bias.reshape(_NBLK, 1, _CH),
        spatial_pos.reshape(_NBLK, 1, _CH).astype(jnp.int32),
        eb_bf.reshape(_NBLK, _CH, _H),
    )
    return out.reshape(_B, _H, _N, _N)


# speedup vs baseline: 1.9377x; 1.9377x over previous
import functools

import jax
import jax.numpy as jnp
from jax import lax
from jax.experimental import pallas as pl
from jax.experimental.pallas import tpu as pltpu
from jax.experimental.pallas import tpu_sc as plsc

_B = 8
_N = 128
_H = 32
_V = 1025
_D = 5
_F = 3
_K = _D * _F
_NPOS = _B * _N * _N
_NW = 32
_P = 128
_NCH = _NPOS // (_NW * _P)
_CH = 2048


def _tables_body(ee_ref, w5_ref, t_ref):
    row = lax.broadcasted_iota(jnp.int32, (_V, _H), 0)
    ee0 = jnp.where(row == 0, 0.0, ee_ref[...])
    for d in range(_D):
        t_ref[d] = jnp.dot(
            ee0, w5_ref[d], preferred_element_type=jnp.float32
        ).astype(jnp.bfloat16)


_tables_call = pl.pallas_call(
    _tables_body,
    out_shape=jax.ShapeDtypeStruct((_D, _V, _H), jnp.bfloat16),
)


def _sc_body(t_hbm, idx_hbm, out_hbm, idx_v, acc_v, sem):
    wid = lax.axis_index("s") * 2 + lax.axis_index("c")

    def chunk_body(c, _):
        pltpu.sync_copy(idx_hbm.at[pl.ds((wid * _NCH + c) * _K, _K)], idx_v)
        for k in range(_F, _K):
            off = (k // _F) * _V
            for j in range(_P // 16):
                sl = pl.ds(j * 16, 16)
                idx_v[k, sl] = idx_v[k, sl] + off
        zero = jnp.zeros((_H,), jnp.bfloat16)

        def zero_body(p, _):
            acc_v[p, :] = zero
            return 0

        lax.fori_loop(0, _P, zero_body, 0)
        descs = [
            pltpu.async_copy(t_hbm.at[idx_v.at[k]], acc_v, sem, add=True)
            for k in range(_K)
        ]
        for desc in descs:
            desc.wait()
        pltpu.sync_copy(acc_v, out_hbm.at[pl.ds(wid * _NCH * _P + c * _P, _P)])
        return 0

    lax.fori_loop(0, _NCH, chunk_body, 0)


@functools.cache
def _sc_call():
    return pl.kernel(
        _sc_body,
        out_type=jax.ShapeDtypeStruct((_NPOS, _H), jnp.bfloat16),
        mesh=plsc.VectorSubcoreMesh(
            core_axis_name="c", subcore_axis_name="s",
            num_cores=2, num_subcores=16,
        ),
        scratch_types=[
            pltpu.VMEM((_K, _P), jnp.int32),
            pltpu.VMEM((_P, _H), jnp.bfloat16),
            pltpu.SemaphoreType.DMA,
        ],
        compiler_params=pltpu.CompilerParams(use_tc_tiling_on_sc=False),
    )


def _finish_body(ab_ref, sp_ref, eb_ref, out_ref):
    spi = sp_ref[0]
    spi = jnp.where(spi == 0, 1, spi)
    spi = jnp.where(spi > 1, spi - 1, spi)
    spf = jnp.clip(spi, 0, _D).astype(jnp.float32)
    scale = 1.0 / (3.0 * spf)
    eye = (
        lax.broadcasted_iota(jnp.int32, (_H, _H), 0)
        == lax.broadcasted_iota(jnp.int32, (_H, _H), 1)
    ).astype(jnp.bfloat16)
    ebt = lax.dot_general(
        eye, eb_ref[0], (((1,), (1,)), ((), ())),
        preferred_element_type=jnp.float32,
    )
    out_ref[0] = ab_ref[0] + ebt * scale


_NBLK = _NPOS // _CH

_finish_call = pl.pallas_call(
    _finish_body,
    grid=(_B, _N * _N // _CH),
    in_specs=[
        pl.BlockSpec((1, 1, _CH), lambda b, c: (b * (_N * _N // _CH) + c, 0, 0)),
        pl.BlockSpec((1, 1, _CH), lambda b, c: (b * (_N * _N // _CH) + c, 0, 0)),
        pl.BlockSpec((1, _CH, _H), lambda b, c: (b * (_N * _N // _CH) + c, 0, 0)),
    ],
    out_specs=pl.BlockSpec((1, _H, _CH), lambda b, c: (b, 0, c)),
    out_shape=jax.ShapeDtypeStruct((_B, _H, _N * _N), jnp.float32),
)


def kernel(attn_bias, spatial_pos, x, attn_edge_type, edge_input,
           edge_encoder_weight, edge_dis_encoder_weight):
    del x, attn_edge_type
    w5 = edge_dis_encoder_weight[: _D * _H * _H].reshape(_D, _H, _H)
    t = _tables_call(edge_encoder_weight, w5).reshape(_D * _V, _H)
    idx4 = (
        edge_input.reshape(_NW, _NCH, _P, _K)
        .transpose(0, 1, 3, 2)
        .astype(jnp.int32)
        .reshape(_NW * _NCH * _K, _P)
    )
    eb = _sc_call()(t, idx4)
    out = _finish_call(
        attn_bias.reshape(_NBLK, 1, _CH),
        spatial_pos.reshape(_NBLK, 1, _CH).astype(jnp.int32),
        eb.reshape(_NBLK, _CH, _H),
    )
    return out.reshape(_B, _H, _N, _N)
